# baseline (speedup 1.0000x reference)
import functools

import jax
import jax.numpy as jnp
from jax import lax
from jax.experimental import pallas as pl
from jax.experimental.pallas import tpu as pltpu
from jax.experimental.pallas import tpu_sc as plsc

_N = 10000
_E = 320000
_D = 128
_HID = 256
_LBL = 10

_NC, _NS = 2, 16
_NW = _NC * _NS
_C = 128
_CR = 2560
_EPAD = _CR * _C - _E
_CPW = _CR // _NW
_G = 8
_GPAIR = _CPW // (2 * _G)

_RB = 80
_NB = _N // _RB
_BPT = -(-_NB // _NS)

_BR = 1000


def _make_agg():
    mesh = plsc.VectorSubcoreMesh(
        core_axis_name="c", subcore_axis_name="s",
        num_cores=_NC, num_subcores=_NS)

    @functools.partial(
        pl.kernel,
        out_type=jax.ShapeDtypeStruct((_NC, _N, _D), jnp.float32),
        mesh=mesh,
        scratch_types=[
            pltpu.VMEM((_G, _C), jnp.int32),
            pltpu.VMEM((_G, _C), jnp.int32),
            pltpu.VMEM((_G, _C), jnp.int32),
            pltpu.VMEM((_G, _C), jnp.int32),
            pltpu.VMEM((_C, _D), jnp.float32),
            pltpu.VMEM((_C, _D), jnp.float32),
            pltpu.VMEM_SHARED((_N + 8, _D), jnp.float32),
            pltpu.SemaphoreType.DMA,
            pltpu.SemaphoreType.DMA,
            pltpu.SemaphoreType.DMA,
        ],
    )
    def agg(h_hbm, src_hbm, dst_hbm, out_hbm,
            srcg0, srcg1, dstg0, dstg1, rows0, rows1,
            acc_s, gsem0, gsem1, isem):
        cid = lax.axis_index("c")
        sid = lax.axis_index("s")
        wid = sid * _NC + cid
        srcg = (srcg0, srcg1)
        dstg = (dstg0, dstg1)
        rows = (rows0, rows1)
        gsem = (gsem0, gsem1)

        for i in range(_BPT):
            b = sid + i * _NS

            @pl.when(b < _NB)
            def _(b=b):
                r = b * _RB
                pltpu.async_copy(h_hbm.at[pl.ds(r, _RB)],
                                 acc_s.at[pl.ds(r, _RB)], isem)
        for i in range(_BPT):
            b = sid + i * _NS

            @pl.when(b < _NB)
            def _(b=b):
                r = b * _RB
                pltpu.make_async_copy(h_hbm.at[pl.ds(r, _RB)],
                                      acc_s.at[pl.ds(r, _RB)], isem).wait()

        plsc.subcore_barrier()

        crow_w = wid * _CPW

        def pair_body(gg, carry):
            cbase = crow_w + gg * (2 * _G)
            last = [None, None]
            for half in range(2):
                gb = cbase + half * _G
                pltpu.sync_copy(src_hbm.at[pl.ds(gb, _G)], srcg[half])
                pltpu.sync_copy(dst_hbm.at[pl.ds(gb, _G)], dstg[half])
                for j in range(_G):
                    k = half * _G + j
                    p = k & 1
                    if last[p] is not None:
                        d, dref = last[p]
                        d.wait()
                        pltpu.sync_copy(rows[p], acc_s.at[dref], add=True)
                    d = pltpu.async_copy(h_hbm.at[srcg[half].at[j]],
                                         rows[p], gsem[p])
                    last[p] = (d, dstg[half].at[j])
            for p in (0, 1):
                d, dref = last[p]
                d.wait()
                pltpu.sync_copy(rows[p], acc_s.at[dref], add=True)
            return carry
        lax.fori_loop(0, _GPAIR, pair_body, 0)

        plsc.subcore_barrier()

        for i in range(_BPT):
            b = sid + i * _NS

            @pl.when(b < _NB)
            def _(b=b):
                r = b * _RB
                pltpu.async_copy(acc_s.at[pl.ds(r, _RB)],
                                 out_hbm.at[cid, pl.ds(r, _RB)], isem)
        for i in range(_BPT):
            b = sid + i * _NS

            @pl.when(b < _NB)
            def _(b=b):
                r = b * _RB
                pltpu.make_async_copy(acc_s.at[pl.ds(r, _RB)],
                                      out_hbm.at[cid, pl.ds(r, _RB)],
                                      isem).wait()

    return agg


_agg = _make_agg()


def _dense(c, h, W, b):
    def body(c_ref, h_ref, W_ref, b_ref, o_ref):
        comb = c_ref[0] + c_ref[1] - h_ref[...]
        o_ref[...] = jnp.maximum(comb @ W_ref[...] + b_ref[...], 0.0)

    return pl.pallas_call(
        body,
        grid=(_N // _BR,),
        in_specs=[
            pl.BlockSpec((2, _BR, _D), lambda i: (0, i, 0)),
            pl.BlockSpec((_BR, _D), lambda i: (i, 0)),
            pl.BlockSpec((_D, _D), lambda i: (0, 0)),
            pl.BlockSpec((1, _D), lambda i: (0, 0)),
        ],
        out_specs=pl.BlockSpec((_BR, _D), lambda i: (i, 0)),
        out_shape=jax.ShapeDtypeStruct((_N, _D), jnp.float32),
    )(c, h, W, b.reshape(1, _D))


def _final(c, h, W2, b2, Wm1, bm1, Wm2, bm2):
    def body(c_ref, h_ref, W2_ref, b2_ref, Wm1_ref, bm1_ref, Wm2_ref,
             bm2_ref, o_ref):
        comb = c_ref[0] + c_ref[1] - h_ref[...]
        h3 = jnp.maximum(comb @ W2_ref[...] + b2_ref[...], 0.0)
        t = jnp.maximum(h3 @ Wm1_ref[...] + bm1_ref[...], 0.0)
        o_ref[...] = t @ Wm2_ref[...] + bm2_ref[...]

    return pl.pallas_call(
        body,
        grid=(_N // _BR,),
        in_specs=[
            pl.BlockSpec((2, _BR, _D), lambda i: (0, i, 0)),
            pl.BlockSpec((_BR, _D), lambda i: (i, 0)),
            pl.BlockSpec((_D, _D), lambda i: (0, 0)),
            pl.BlockSpec((1, _D), lambda i: (0, 0)),
            pl.BlockSpec((_D, _HID), lambda i: (0, 0)),
            pl.BlockSpec((1, _HID), lambda i: (0, 0)),
            pl.BlockSpec((_HID, _LBL), lambda i: (0, 0)),
            pl.BlockSpec((1, _LBL), lambda i: (0, 0)),
        ],
        out_specs=pl.BlockSpec((_BR, _LBL), lambda i: (i, 0)),
        out_shape=jax.ShapeDtypeStruct((_N, _LBL), jnp.float32),
    )(c, h, W2, b2.reshape(1, _D), Wm1, bm1.reshape(1, _HID),
      Wm2, bm2.reshape(1, _LBL))


def kernel(x, edge_index, edge_weight, W0, b0, W1, b1, W2, b2,
           Wm1, bm1, Wm2, bm2):
    del edge_weight
    src = jnp.concatenate(
        [edge_index[0], jnp.zeros((_EPAD,), jnp.int32)]).reshape(_CR, _C)
    dst = jnp.concatenate(
        [edge_index[1], jnp.full((_EPAD,), _N, jnp.int32)]).reshape(_CR, _C)

    c1 = _agg(x, src, dst)
    h1 = _dense(c1, x, W0, b0)
    c2 = _agg(h1, src, dst)
    h2 = _dense(c2, h1, W1, b1)
    c3 = _agg(h2, src, dst)
    return _final(c3, h2, W2, b2, Wm1, bm1, Wm2, bm2)


# speedup vs baseline: 1.0367x; 1.0367x over previous
import functools

import jax
import jax.numpy as jnp
from jax import lax
from jax.experimental import pallas as pl
from jax.experimental.pallas import tpu as pltpu
from jax.experimental.pallas import tpu_sc as plsc

_N = 10000
_E = 320000
_D = 128
_HID = 256
_LBL = 10

_NC, _NS = 2, 16
_NW = _NC * _NS
_C = 128
_CR = 2560
_EPAD = _CR * _C - _E
_CPW = _CR // _NW
_G = 8
_GPAIR = _CPW // (2 * _G)

_RB = 80
_NB = _N // _RB
_BPT = -(-_NB // _NS)

_BR = 1000


def _make_agg():
    mesh = plsc.VectorSubcoreMesh(
        core_axis_name="c", subcore_axis_name="s",
        num_cores=_NC, num_subcores=_NS)

    @functools.partial(
        pl.kernel,
        out_type=jax.ShapeDtypeStruct((_NC, _N, _D), jnp.float32),
        mesh=mesh,
        scratch_types=[
            pltpu.VMEM((_G, _C), jnp.int32),
            pltpu.VMEM((_G, _C), jnp.int32),
            pltpu.VMEM((_G, _C), jnp.int32),
            pltpu.VMEM((_G, _C), jnp.int32),
            pltpu.VMEM((_C, _D), jnp.float32),
            pltpu.VMEM((_C, _D), jnp.float32),
            pltpu.VMEM_SHARED((_N, _D), jnp.float32),
            pltpu.SemaphoreType.DMA,
            pltpu.SemaphoreType.DMA,
            pltpu.SemaphoreType.DMA,
        ],
    )
    def agg(h_hbm, src_hbm, dst_hbm, out_hbm,
            srcg0, srcg1, dstg0, dstg1, rows0, rows1,
            acc_s, gsem0, gsem1, isem):
        cid = lax.axis_index("c")
        sid = lax.axis_index("s")
        wid = sid * _NC + cid
        srcg = (srcg0, srcg1)
        dstg = (dstg0, dstg1)
        rows = (rows0, rows1)
        gsem = (gsem0, gsem1)

        for i in range(_BPT):
            b = sid + i * _NS

            @pl.when(b < _NB)
            def _(b=b):
                r = b * _RB
                pltpu.async_copy(h_hbm.at[pl.ds(r, _RB)],
                                 acc_s.at[pl.ds(r, _RB)], isem)
        for i in range(_BPT):
            b = sid + i * _NS

            @pl.when(b < _NB)
            def _(b=b):
                r = b * _RB
                pltpu.make_async_copy(h_hbm.at[pl.ds(r, _RB)],
                                      acc_s.at[pl.ds(r, _RB)], isem).wait()

        plsc.subcore_barrier()

        crow_w = wid * _CPW

        def pair_body(gg, carry):
            cbase = crow_w + gg * (2 * _G)
            last = [None, None]
            for half in range(2):
                gb = cbase + half * _G
                pltpu.sync_copy(src_hbm.at[pl.ds(gb, _G)], srcg[half])
                pltpu.sync_copy(dst_hbm.at[pl.ds(gb, _G)], dstg[half])
                for j in range(_G):
                    k = half * _G + j
                    p = k & 1
                    if last[p] is not None:
                        d, dref = last[p]
                        d.wait()
                        pltpu.sync_copy(rows[p], acc_s.at[dref], add=True)
                    d = pltpu.async_copy(h_hbm.at[srcg[half].at[j]],
                                         rows[p], gsem[p])
                    last[p] = (d, dstg[half].at[j])
            for p in (0, 1):
                d, dref = last[p]
                d.wait()
                pltpu.sync_copy(rows[p], acc_s.at[dref], add=True)
            return carry
        lax.fori_loop(0, _GPAIR, pair_body, 0)

        plsc.subcore_barrier()

        for i in range(_BPT):
            b = sid + i * _NS

            @pl.when(b < _NB)
            def _(b=b):
                r = b * _RB
                pltpu.async_copy(acc_s.at[pl.ds(r, _RB)],
                                 out_hbm.at[cid, pl.ds(r, _RB)], isem)
        for i in range(_BPT):
            b = sid + i * _NS

            @pl.when(b < _NB)
            def _(b=b):
                r = b * _RB
                pltpu.make_async_copy(acc_s.at[pl.ds(r, _RB)],
                                      out_hbm.at[cid, pl.ds(r, _RB)],
                                      isem).wait()

    return agg


_agg = _make_agg()


def _dense(c, h, W, b):
    def body(c_ref, h_ref, W_ref, b_ref, o_ref):
        comb = c_ref[0] + c_ref[1] - h_ref[...]
        o_ref[...] = jnp.maximum(comb @ W_ref[...] + b_ref[...], 0.0)

    return pl.pallas_call(
        body,
        grid=(_N // _BR,),
        in_specs=[
            pl.BlockSpec((2, _BR, _D), lambda i: (0, i, 0)),
            pl.BlockSpec((_BR, _D), lambda i: (i, 0)),
            pl.BlockSpec((_D, _D), lambda i: (0, 0)),
            pl.BlockSpec((1, _D), lambda i: (0, 0)),
        ],
        out_specs=pl.BlockSpec((_BR, _D), lambda i: (i, 0)),
        out_shape=jax.ShapeDtypeStruct((_N, _D), jnp.float32),
    )(c, h, W, b.reshape(1, _D))


def _final(c, h, W2, b2, Wm1, bm1, Wm2, bm2):
    def body(c_ref, h_ref, W2_ref, b2_ref, Wm1_ref, bm1_ref, Wm2_ref,
             bm2_ref, o_ref):
        comb = c_ref[0] + c_ref[1] - h_ref[...]
        h3 = jnp.maximum(comb @ W2_ref[...] + b2_ref[...], 0.0)
        t = jnp.maximum(h3 @ Wm1_ref[...] + bm1_ref[...], 0.0)
        o_ref[...] = t @ Wm2_ref[...] + bm2_ref[...]

    return pl.pallas_call(
        body,
        grid=(_N // _BR,),
        in_specs=[
            pl.BlockSpec((2, _BR, _D), lambda i: (0, i, 0)),
            pl.BlockSpec((_BR, _D), lambda i: (i, 0)),
            pl.BlockSpec((_D, _D), lambda i: (0, 0)),
            pl.BlockSpec((1, _D), lambda i: (0, 0)),
            pl.BlockSpec((_D, _HID), lambda i: (0, 0)),
            pl.BlockSpec((1, _HID), lambda i: (0, 0)),
            pl.BlockSpec((_HID, _LBL), lambda i: (0, 0)),
            pl.BlockSpec((1, _LBL), lambda i: (0, 0)),
        ],
        out_specs=pl.BlockSpec((_BR, _LBL), lambda i: (i, 0)),
        out_shape=jax.ShapeDtypeStruct((_N, _LBL), jnp.float32),
    )(c, h, W2, b2.reshape(1, _D), Wm1, bm1.reshape(1, _HID),
      Wm2, bm2.reshape(1, _LBL))


def kernel(x, edge_index, edge_weight, W0, b0, W1, b1, W2, b2,
           Wm1, bm1, Wm2, bm2):
    del edge_weight
    src = jnp.concatenate(
        [edge_index[0], jnp.full((_EPAD,), _N, jnp.int32)]).reshape(_CR, _C)
    dst = jnp.concatenate(
        [edge_index[1],
         jnp.arange(_EPAD, dtype=jnp.int32) % _N]).reshape(_CR, _C)
    zpad = jnp.zeros((8, _D), jnp.float32)

    c1 = _agg(jnp.concatenate([x, zpad]), src, dst)
    h1 = _dense(c1, x, W0, b0)
    c2 = _agg(jnp.concatenate([h1, zpad]), src, dst)
    h2 = _dense(c2, h1, W1, b1)
    c3 = _agg(jnp.concatenate([h2, zpad]), src, dst)
    return _final(c3, h2, W2, b2, Wm1, bm1, Wm2, bm2)


# speedup vs baseline: 3.4680x; 3.3453x over previous
import functools

import jax
import jax.numpy as jnp
from jax import lax
from jax.experimental import pallas as pl
from jax.experimental.pallas import tpu as pltpu
from jax.experimental.pallas import tpu_sc as plsc

_N = 10000
_E = 320000
_D = 128
_HID = 256
_LBL = 10

_NC, _NS = 2, 16
_NW = _NC * _NS
_C = 128
_CR = 2560
_EPAD = _CR * _C - _E
_CPW = _CR // _NW
_G = 8
_GPAIR = _CPW // (2 * _G)

_RB = 80
_NB = _N // _RB
_BPT = -(-_NB // _NS)

_BR = 1000


def _make_agg():
    mesh = plsc.VectorSubcoreMesh(
        core_axis_name="c", subcore_axis_name="s",
        num_cores=_NC, num_subcores=_NS)

    @functools.partial(
        pl.kernel,
        out_type=jax.ShapeDtypeStruct((_NC, _N, _D), jnp.float32),
        mesh=mesh,
        scratch_types=[
            pltpu.VMEM((_G, _C), jnp.int32),
            pltpu.VMEM((_G, _C), jnp.int32),
            pltpu.VMEM((_G, _C), jnp.int32),
            pltpu.VMEM((_G, _C), jnp.int32),
            pltpu.VMEM((_C, _D), jnp.float32),
            pltpu.VMEM((_C, _D), jnp.float32),
            pltpu.VMEM_SHARED((_N + 64, _D), jnp.float32),
            pltpu.SemaphoreType.DMA,
            pltpu.SemaphoreType.DMA,
            pltpu.SemaphoreType.DMA,
        ],
    )
    def agg(h_hbm, src_hbm, dst_hbm, out_hbm,
            srcg0, srcg1, dstg0, dstg1, rows0, rows1,
            acc_s, gsem0, gsem1, isem):
        cid = lax.axis_index("c")
        sid = lax.axis_index("s")
        wid = sid * _NC + cid
        srcg = (srcg0, srcg1)
        dstg = (dstg0, dstg1)
        rows = (rows0, rows1)
        gsem = (gsem0, gsem1)

        for i in range(_BPT):
            b = sid + i * _NS

            @pl.when(b < _NB)
            def _(b=b):
                r = b * _RB
                pltpu.async_copy(h_hbm.at[pl.ds(r, _RB)],
                                 acc_s.at[pl.ds(r, _RB)], isem)
        for i in range(_BPT):
            b = sid + i * _NS

            @pl.when(b < _NB)
            def _(b=b):
                r = b * _RB
                pltpu.make_async_copy(h_hbm.at[pl.ds(r, _RB)],
                                      acc_s.at[pl.ds(r, _RB)], isem).wait()

        plsc.subcore_barrier()

        crow_w = wid * _CPW

        def pair_body(gg, carry):
            cbase = crow_w + gg * (2 * _G)
            last = [None, None]
            for half in range(2):
                gb = cbase + half * _G
                pltpu.sync_copy(src_hbm.at[pl.ds(gb, _G)], srcg[half])
                pltpu.sync_copy(dst_hbm.at[pl.ds(gb, _G)], dstg[half])
                for j in range(_G):
                    k = half * _G + j
                    p = k & 1
                    if last[p] is not None:
                        d, dref = last[p]
                        d.wait()
                        pltpu.sync_copy(rows[p], acc_s.at[dref], add=True)
                    d = pltpu.async_copy(h_hbm.at[srcg[half].at[j]],
                                         rows[p], gsem[p])
                    last[p] = (d, dstg[half].at[j])
            for p in (0, 1):
                d, dref = last[p]
                d.wait()
                pltpu.sync_copy(rows[p], acc_s.at[dref], add=True)
            return carry
        lax.fori_loop(0, _GPAIR, pair_body, 0)

        plsc.subcore_barrier()

        for i in range(_BPT):
            b = sid + i * _NS

            @pl.when(b < _NB)
            def _(b=b):
                r = b * _RB
                pltpu.async_copy(acc_s.at[pl.ds(r, _RB)],
                                 out_hbm.at[cid, pl.ds(r, _RB)], isem)
        for i in range(_BPT):
            b = sid + i * _NS

            @pl.when(b < _NB)
            def _(b=b):
                r = b * _RB
                pltpu.make_async_copy(acc_s.at[pl.ds(r, _RB)],
                                      out_hbm.at[cid, pl.ds(r, _RB)],
                                      isem).wait()

    return agg


_agg = _make_agg()


def _dense(c, h, W, b):
    def body(c_ref, h_ref, W_ref, b_ref, o_ref):
        comb = c_ref[0] + c_ref[1] - h_ref[...]
        o_ref[...] = jnp.maximum(comb @ W_ref[...] + b_ref[...], 0.0)

    return pl.pallas_call(
        body,
        grid=(_N // _BR,),
        in_specs=[
            pl.BlockSpec((2, _BR, _D), lambda i: (0, i, 0)),
            pl.BlockSpec((_BR, _D), lambda i: (i, 0)),
            pl.BlockSpec((_D, _D), lambda i: (0, 0)),
            pl.BlockSpec((1, _D), lambda i: (0, 0)),
        ],
        out_specs=pl.BlockSpec((_BR, _D), lambda i: (i, 0)),
        out_shape=jax.ShapeDtypeStruct((_N, _D), jnp.float32),
    )(c, h, W, b.reshape(1, _D))


def _final(c, h, W2, b2, Wm1, bm1, Wm2, bm2):
    def body(c_ref, h_ref, W2_ref, b2_ref, Wm1_ref, bm1_ref, Wm2_ref,
             bm2_ref, o_ref):
        comb = c_ref[0] + c_ref[1] - h_ref[...]
        h3 = jnp.maximum(comb @ W2_ref[...] + b2_ref[...], 0.0)
        t = jnp.maximum(h3 @ Wm1_ref[...] + bm1_ref[...], 0.0)
        o_ref[...] = t @ Wm2_ref[...] + bm2_ref[...]

    return pl.pallas_call(
        body,
        grid=(_N // _BR,),
        in_specs=[
            pl.BlockSpec((2, _BR, _D), lambda i: (0, i, 0)),
            pl.BlockSpec((_BR, _D), lambda i: (i, 0)),
            pl.BlockSpec((_D, _D), lambda i: (0, 0)),
            pl.BlockSpec((1, _D), lambda i: (0, 0)),
            pl.BlockSpec((_D, _HID), lambda i: (0, 0)),
            pl.BlockSpec((1, _HID), lambda i: (0, 0)),
            pl.BlockSpec((_HID, _LBL), lambda i: (0, 0)),
            pl.BlockSpec((1, _LBL), lambda i: (0, 0)),
        ],
        out_specs=pl.BlockSpec((_BR, _LBL), lambda i: (i, 0)),
        out_shape=jax.ShapeDtypeStruct((_N, _LBL), jnp.float32),
    )(c, h, W2, b2.reshape(1, _D), Wm1, bm1.reshape(1, _HID),
      Wm2, bm2.reshape(1, _LBL))


def kernel(x, edge_index, edge_weight, W0, b0, W1, b1, W2, b2,
           Wm1, bm1, Wm2, bm2):
    del edge_weight
    pad_ar = jnp.arange(_EPAD, dtype=jnp.int32)
    src = jnp.concatenate(
        [edge_index[0], pad_ar % _N]).reshape(_CR, _C)
    dst = jnp.concatenate(
        [edge_index[1], _N + (pad_ar % 64)]).reshape(_CR, _C)

    c1 = _agg(x, src, dst)
    h1 = _dense(c1, x, W0, b0)
    c2 = _agg(h1, src, dst)
    h2 = _dense(c2, h1, W1, b1)
    c3 = _agg(h2, src, dst)
    return _final(c3, h2, W2, b2, Wm1, bm1, Wm2, bm2)


# speedup vs baseline: 3.6602x; 1.0554x over previous
import functools

import jax
import jax.numpy as jnp
from jax import lax
from jax.experimental import pallas as pl
from jax.experimental.pallas import tpu as pltpu
from jax.experimental.pallas import tpu_sc as plsc

_N = 10000
_E = 320000
_D = 128
_HID = 256
_LBL = 10

_NC, _NS = 2, 16
_NW = _NC * _NS
_C = 128
_CR = 2560
_EPAD = _CR * _C - _E
_CPW = _CR // _NW
_G = 8
_GPAIR = _CPW // (2 * _G)

_RB = 80
_NB = _N // _RB
_BPT = -(-_NB // _NS)

_BR = 1000


def _make_agg():
    mesh = plsc.VectorSubcoreMesh(
        core_axis_name="c", subcore_axis_name="s",
        num_cores=_NC, num_subcores=_NS)

    @functools.partial(
        pl.kernel,
        out_type=jax.ShapeDtypeStruct((_NC, _N, _D), jnp.float32),
        mesh=mesh,
        scratch_types=[
            pltpu.VMEM((_G, _C), jnp.int32),
            pltpu.VMEM((_G, _C), jnp.int32),
            pltpu.VMEM((_G, _C), jnp.int32),
            pltpu.VMEM((_G, _C), jnp.int32),
            pltpu.VMEM((_C, _D), jnp.float32),
            pltpu.VMEM((_C, _D), jnp.float32),
            pltpu.VMEM_SHARED((_N + 64, _D), jnp.float32),
            pltpu.SemaphoreType.DMA,
            pltpu.SemaphoreType.DMA,
            pltpu.SemaphoreType.DMA,
            pltpu.SemaphoreType.DMA,
            pltpu.SemaphoreType.DMA,
        ],
    )
    def agg(h_hbm, src_hbm, dst_hbm, out_hbm,
            srcg0, srcg1, dstg0, dstg1, rows0, rows1,
            acc_s, gsem0, gsem1, isem, isemA, isemB):
        cid = lax.axis_index("c")
        sid = lax.axis_index("s")
        wid = sid * _NC + cid
        srcg = (srcg0, srcg1)
        dstg = (dstg0, dstg1)
        rows = (rows0, rows1)
        gsem = (gsem0, gsem1)

        for i in range(_BPT):
            b = sid + i * _NS

            @pl.when(b < _NB)
            def _(b=b):
                r = b * _RB
                pltpu.async_copy(h_hbm.at[pl.ds(r, _RB)],
                                 acc_s.at[pl.ds(r, _RB)], isem)
        for i in range(_BPT):
            b = sid + i * _NS

            @pl.when(b < _NB)
            def _(b=b):
                r = b * _RB
                pltpu.make_async_copy(h_hbm.at[pl.ds(r, _RB)],
                                      acc_s.at[pl.ds(r, _RB)], isem).wait()

        plsc.subcore_barrier()

        crow_w = wid * _CPW

        def _drain(rows_p, gsem_p, dref):
            pltpu.make_async_copy(h_hbm.at[pl.ds(0, _C)], rows_p,
                                  gsem_p).wait()
            pltpu.sync_copy(rows_p, acc_s.at[dref], add=True)

        def pair_body(gg, carry):
            cbase = crow_w + gg * (2 * _G)

            @pl.when(gg == 0)
            def _():
                pltpu.sync_copy(src_hbm.at[pl.ds(cbase, _G)], srcg0)
                pltpu.sync_copy(dst_hbm.at[pl.ds(cbase, _G)], dstg0)

            @pl.when(gg > 0)
            def _():
                pltpu.make_async_copy(src_hbm.at[pl.ds(cbase, _G)],
                                      srcg0, isemA).wait()
                pltpu.make_async_copy(dst_hbm.at[pl.ds(cbase, _G)],
                                      dstg0, isemA).wait()
                _drain(rows0, gsem0, dstg1.at[_G - 2])
                _drain(rows1, gsem1, dstg1.at[_G - 1])

            pltpu.async_copy(src_hbm.at[pl.ds(cbase + _G, _G)],
                             srcg1, isemB)
            pltpu.async_copy(dst_hbm.at[pl.ds(cbase + _G, _G)],
                             dstg1, isemB)

            last = [None, None]
            for k in range(2 * _G):
                half, j = divmod(k, _G)
                p = k & 1
                if k == _G:
                    pltpu.make_async_copy(src_hbm.at[pl.ds(cbase + _G, _G)],
                                          srcg1, isemB).wait()
                    pltpu.make_async_copy(dst_hbm.at[pl.ds(cbase + _G, _G)],
                                          dstg1, isemB).wait()
                if last[p] is not None:
                    d, dref = last[p]
                    d.wait()
                    pltpu.sync_copy(rows[p], acc_s.at[dref], add=True)
                d = pltpu.async_copy(h_hbm.at[srcg[half].at[j]],
                                     rows[p], gsem[p])
                last[p] = (d, dstg[half].at[j])

            @pl.when(gg < _GPAIR - 1)
            def _():
                pltpu.async_copy(src_hbm.at[pl.ds(cbase + 2 * _G, _G)],
                                 srcg0, isemA)
                pltpu.async_copy(dst_hbm.at[pl.ds(cbase + 2 * _G, _G)],
                                 dstg0, isemA)
            return carry
        lax.fori_loop(0, _GPAIR, pair_body, 0)

        _drain(rows0, gsem0, dstg1.at[_G - 2])
        _drain(rows1, gsem1, dstg1.at[_G - 1])

        plsc.subcore_barrier()

        for i in range(_BPT):
            b = sid + i * _NS

            @pl.when(b < _NB)
            def _(b=b):
                r = b * _RB
                pltpu.async_copy(acc_s.at[pl.ds(r, _RB)],
                                 out_hbm.at[cid, pl.ds(r, _RB)], isem)
        for i in range(_BPT):
            b = sid + i * _NS

            @pl.when(b < _NB)
            def _(b=b):
                r = b * _RB
                pltpu.make_async_copy(acc_s.at[pl.ds(r, _RB)],
                                      out_hbm.at[cid, pl.ds(r, _RB)],
                                      isem).wait()

    return agg


_agg = _make_agg()


def _dense(c, h, W, b):
    def body(c_ref, h_ref, W_ref, b_ref, o_ref):
        comb = c_ref[0] + c_ref[1] - h_ref[...]
        o_ref[...] = jnp.maximum(comb @ W_ref[...] + b_ref[...], 0.0)

    return pl.pallas_call(
        body,
        grid=(_N // _BR,),
        in_specs=[
            pl.BlockSpec((2, _BR, _D), lambda i: (0, i, 0)),
            pl.BlockSpec((_BR, _D), lambda i: (i, 0)),
            pl.BlockSpec((_D, _D), lambda i: (0, 0)),
            pl.BlockSpec((1, _D), lambda i: (0, 0)),
        ],
        out_specs=pl.BlockSpec((_BR, _D), lambda i: (i, 0)),
        out_shape=jax.ShapeDtypeStruct((_N, _D), jnp.float32),
    )(c, h, W, b.reshape(1, _D))


def _final(c, h, W2, b2, Wm1, bm1, Wm2, bm2):
    def body(c_ref, h_ref, W2_ref, b2_ref, Wm1_ref, bm1_ref, Wm2_ref,
             bm2_ref, o_ref):
        comb = c_ref[0] + c_ref[1] - h_ref[...]
        h3 = jnp.maximum(comb @ W2_ref[...] + b2_ref[...], 0.0)
        t = jnp.maximum(h3 @ Wm1_ref[...] + bm1_ref[...], 0.0)
        o_ref[...] = t @ Wm2_ref[...] + bm2_ref[...]

    return pl.pallas_call(
        body,
        grid=(_N // _BR,),
        in_specs=[
            pl.BlockSpec((2, _BR, _D), lambda i: (0, i, 0)),
            pl.BlockSpec((_BR, _D), lambda i: (i, 0)),
            pl.BlockSpec((_D, _D), lambda i: (0, 0)),
            pl.BlockSpec((1, _D), lambda i: (0, 0)),
            pl.BlockSpec((_D, _HID), lambda i: (0, 0)),
            pl.BlockSpec((1, _HID), lambda i: (0, 0)),
            pl.BlockSpec((_HID, _LBL), lambda i: (0, 0)),
            pl.BlockSpec((1, _LBL), lambda i: (0, 0)),
        ],
        out_specs=pl.BlockSpec((_BR, _LBL), lambda i: (i, 0)),
        out_shape=jax.ShapeDtypeStruct((_N, _LBL), jnp.float32),
    )(c, h, W2, b2.reshape(1, _D), Wm1, bm1.reshape(1, _HID),
      Wm2, bm2.reshape(1, _LBL))


def kernel(x, edge_index, edge_weight, W0, b0, W1, b1, W2, b2,
           Wm1, bm1, Wm2, bm2):
    del edge_weight
    pad_ar = jnp.arange(_EPAD, dtype=jnp.int32)
    src = jnp.concatenate(
        [edge_index[0], pad_ar % _N]).reshape(_CR, _C)
    dst = jnp.concatenate(
        [edge_index[1], _N + (pad_ar % 64)]).reshape(_CR, _C)

    c1 = _agg(x, src, dst)
    h1 = _dense(c1, x, W0, b0)
    c2 = _agg(h1, src, dst)
    h2 = _dense(c2, h1, W1, b1)
    c3 = _agg(h2, src, dst)
    return _final(c3, h2, W2, b2, Wm1, bm1, Wm2, bm2)
